# baseline (device time: 8304 ns/iter reference)
import jax
import jax.numpy as jnp
from jax import lax
from jax.experimental import pallas as pl
from jax.experimental.pallas import tpu as pltpu


def kernel(x):
    m, n = x.shape
    half = n // 2

    def body(x_ref, out_ref, send_sem, recv_sem, send_sem2, recv_sem2):
        my_x = lax.axis_index("x")
        my_y = lax.axis_index("y")
        peer = 1 - my_y

        barrier_sem = pltpu.get_barrier_semaphore()
        pl.semaphore_signal(
            barrier_sem, inc=1,
            device_id=(my_x, peer), device_id_type=pl.DeviceIdType.MESH,
        )

        out_ref[pl.ds(my_y * m, m), :] = x_ref[:, pl.ds(my_y * half, half)]

        pl.semaphore_wait(barrier_sem, 1)

        hm = m // 2
        rdma0 = pltpu.make_async_remote_copy(
            src_ref=x_ref.at[pl.ds(0, hm), pl.ds(peer * half, half)],
            dst_ref=out_ref.at[pl.ds(my_y * m, hm), :],
            send_sem=send_sem,
            recv_sem=recv_sem,
            device_id=(my_x, peer),
            device_id_type=pl.DeviceIdType.MESH,
        )
        rdma1 = pltpu.make_async_remote_copy(
            src_ref=x_ref.at[pl.ds(hm, hm), pl.ds(peer * half, half)],
            dst_ref=out_ref.at[pl.ds(my_y * m + hm, hm), :],
            send_sem=send_sem2,
            recv_sem=recv_sem2,
            device_id=(my_x, peer),
            device_id_type=pl.DeviceIdType.MESH,
        )
        rdma0.start()
        rdma1.start()

        rdma0.wait()
        rdma1.wait()

    out_shape = jax.ShapeDtypeStruct((2 * m, half), x.dtype)
    return pl.pallas_call(
        body,
        out_shape=out_shape,
        in_specs=[pl.BlockSpec(memory_space=pltpu.VMEM)],
        out_specs=pl.BlockSpec(memory_space=pltpu.VMEM),
        scratch_shapes=[
            pltpu.SemaphoreType.DMA,
            pltpu.SemaphoreType.DMA,
            pltpu.SemaphoreType.DMA,
            pltpu.SemaphoreType.DMA,
        ],
        compiler_params=pltpu.CompilerParams(collective_id=0),
    )(x)


# device time: 8275 ns/iter; 1.0035x vs baseline; 1.0035x over previous
import jax
import jax.numpy as jnp
from jax import lax
from jax.experimental import pallas as pl
from jax.experimental.pallas import tpu as pltpu


def kernel(x):
    m, n = x.shape
    half = n // 2

    def body(x_ref, out_ref, send_sem, recv_sem, local_sem):
        my_x = lax.axis_index("x")
        my_y = lax.axis_index("y")
        peer = 1 - my_y

        local = pltpu.make_async_copy(
            x_ref.at[:, pl.ds(my_y * half, half)],
            out_ref.at[pl.ds(my_y * m, m), :],
            local_sem,
        )
        local.start()

        barrier_sem = pltpu.get_barrier_semaphore()
        pl.semaphore_signal(
            barrier_sem, inc=1,
            device_id=(my_x, peer), device_id_type=pl.DeviceIdType.MESH,
        )
        pl.semaphore_wait(barrier_sem, 1)

        rdma = pltpu.make_async_remote_copy(
            src_ref=x_ref.at[:, pl.ds(peer * half, half)],
            dst_ref=out_ref.at[pl.ds(my_y * m, m), :],
            send_sem=send_sem,
            recv_sem=recv_sem,
            device_id=(my_x, peer),
            device_id_type=pl.DeviceIdType.MESH,
        )
        rdma.start()

        local.wait()
        rdma.wait()

    out_shape = jax.ShapeDtypeStruct((2 * m, half), x.dtype)
    return pl.pallas_call(
        body,
        out_shape=out_shape,
        in_specs=[pl.BlockSpec(memory_space=pl.ANY)],
        out_specs=pl.BlockSpec(memory_space=pl.ANY),
        scratch_shapes=[
            pltpu.SemaphoreType.DMA,
            pltpu.SemaphoreType.DMA,
            pltpu.SemaphoreType.DMA,
        ],
        compiler_params=pltpu.CompilerParams(collective_id=0),
    )(x)
